# baseline (speedup 1.0000x reference)
import functools

import jax
import jax.numpy as jnp
from jax import lax
from jax.experimental import pallas as pl
from jax.experimental.pallas import tpu as pltpu
from jax.experimental.pallas import tpu_sc as plsc

B = 1024
S = 77
SP = 80
D = 768
L = 16
NC, NS = 2, 16
NW = NC * NS
SEQ_PER_W = B // NW

_mesh = plsc.VectorSubcoreMesh(core_axis_name="c", subcore_axis_name="s")


@functools.partial(
    pl.kernel,
    mesh=_mesh,
    compiler_params=pltpu.CompilerParams(use_tc_tiling_on_sc=False),
    out_type=jax.ShapeDtypeStruct((B * S, D), jnp.float32),
    scratch_types=[
        pltpu.VMEM((SEQ_PER_W * SP,), jnp.int32),
        pltpu.VMEM((SP,), jnp.int32),
        pltpu.VMEM((S, D), jnp.float32),
        pltpu.VMEM((S, D), jnp.float32),
        pltpu.SemaphoreType.DMA,
    ],
)
def _emb_kernel(ids_hbm, pids_hbm, tok_hbm, pos_hbm, out_hbm,
                idx_v, pidx_v, pos_v, buf_v, sem):
    wid = lax.axis_index("s") * NC + lax.axis_index("c")

    pltpu.sync_copy(ids_hbm.at[pl.ds(wid * (SEQ_PER_W * SP), SEQ_PER_W * SP)],
                    idx_v)
    pltpu.sync_copy(pids_hbm, pidx_v)
    pltpu.async_copy(pos_hbm.at[pidx_v.at[pl.ds(0, S)]], pos_v, sem).wait()

    def seq_body(q, _):
        base = wid * (SEQ_PER_W * S) + q * S
        pltpu.async_copy(tok_hbm.at[idx_v.at[pl.ds(q * SP, S)]], buf_v,
                         sem).wait()

        def row_body(r, _):
            for c in range(D // L):
                sl = pl.ds(c * L, L)
                buf_v[r, sl] = buf_v[r, sl] + pos_v[r, sl]
            return 0

        lax.fori_loop(0, S, row_body, 0)
        pltpu.sync_copy(buf_v, out_hbm.at[pl.ds(base, S)])
        return 0

    lax.fori_loop(0, SEQ_PER_W, seq_body, 0)


def kernel(input_ids, position_ids, token_table, pos_table):
    ids = input_ids.astype(jnp.int32).reshape(B, S)
    ids_pad = jnp.pad(ids, ((0, 0), (0, SP - S))).reshape(-1)
    pids = jnp.pad(position_ids.astype(jnp.int32).reshape(-1), (0, SP - S))
    out = _emb_kernel(ids_pad, pids, token_table, pos_table)
    return out.reshape(B, S, D)


# speedup vs baseline: 1.0560x; 1.0560x over previous
import functools

import jax
import jax.numpy as jnp
from jax import lax
from jax.experimental import pallas as pl
from jax.experimental.pallas import tpu as pltpu
from jax.experimental.pallas import tpu_sc as plsc

B = 1024
S = 77
SP = 80
D = 768
L = 16
NC, NS = 2, 16
NW = NC * NS
SEQ_PER_W = B // NW
H0, H1 = 40, 37

_mesh = plsc.VectorSubcoreMesh(core_axis_name="c", subcore_axis_name="s")


@functools.partial(
    pl.kernel,
    mesh=_mesh,
    compiler_params=pltpu.CompilerParams(use_tc_tiling_on_sc=False),
    out_type=jax.ShapeDtypeStruct((B * S, D), jnp.float32),
    scratch_types=[
        pltpu.VMEM((SEQ_PER_W * SP,), jnp.int32),
        pltpu.VMEM((SP,), jnp.int32),
        pltpu.VMEM((S, D), jnp.float32),
        pltpu.VMEM((H0, D), jnp.float32),
        pltpu.VMEM((H1, D), jnp.float32),
        pltpu.SemaphoreType.DMA,
        pltpu.SemaphoreType.DMA,
        pltpu.SemaphoreType.DMA,
        pltpu.SemaphoreType.DMA,
    ],
)
def _emb_kernel(ids_hbm, pids_hbm, tok_hbm, pos_hbm, out_hbm,
                idx_v, pidx_v, pos_v, buf0, buf1, gsem0, gsem1, osem0, osem1):
    wid = lax.axis_index("s") * NC + lax.axis_index("c")

    pltpu.sync_copy(ids_hbm.at[pl.ds(wid * (SEQ_PER_W * SP), SEQ_PER_W * SP)],
                    idx_v)
    pltpu.sync_copy(pids_hbm, pidx_v)
    pltpu.async_copy(pos_hbm.at[pidx_v.at[pl.ds(0, S)]], pos_v, gsem0).wait()

    def add_pos(buf, rows, pos_off):
        def row_body(r, _):
            for c in range(D // L):
                sl = pl.ds(c * L, L)
                plsc.addupdate(buf.at[r, sl], pos_v[pos_off + r, sl])
            return 0
        lax.fori_loop(0, rows, row_body, 0)

    wbase = wid * (SEQ_PER_W * S)

    def seq_body(q, _):
        base = wbase + q * S
        @pl.when(q > 0)
        def _():
            pbase = base - S
            pltpu.make_async_copy(
                buf0, out_hbm.at[pl.ds(pbase, H0)], osem0).wait()
            pltpu.make_async_copy(
                buf1, out_hbm.at[pl.ds(pbase + H0, H1)], osem1).wait()

        g0 = pltpu.async_copy(tok_hbm.at[idx_v.at[pl.ds(q * SP, H0)]],
                              buf0, gsem0)
        g1 = pltpu.async_copy(tok_hbm.at[idx_v.at[pl.ds(q * SP + H0, H1)]],
                              buf1, gsem1)
        g0.wait()
        add_pos(buf0, H0, 0)
        pltpu.async_copy(buf0, out_hbm.at[pl.ds(base, H0)], osem0)
        g1.wait()
        add_pos(buf1, H1, H0)
        pltpu.async_copy(buf1, out_hbm.at[pl.ds(base + H0, H1)], osem1)
        return 0

    lax.fori_loop(0, SEQ_PER_W, seq_body, 0)

    fbase = wbase + (SEQ_PER_W - 1) * S
    pltpu.make_async_copy(buf0, out_hbm.at[pl.ds(fbase, H0)], osem0).wait()
    pltpu.make_async_copy(buf1, out_hbm.at[pl.ds(fbase + H0, H1)], osem1).wait()


def kernel(input_ids, position_ids, token_table, pos_table):
    ids = input_ids.astype(jnp.int32).reshape(B, S)
    ids_pad = jnp.pad(ids, ((0, 0), (0, SP - S))).reshape(-1)
    pids = jnp.pad(position_ids.astype(jnp.int32).reshape(-1), (0, SP - S))
    out = _emb_kernel(ids_pad, pids, token_table, pos_table)
    return out.reshape(B, S, D)
